# baseline (speedup 1.0000x reference)
import functools

import jax
import jax.numpy as jnp
from jax import lax
from jax.experimental import pallas as pl
from jax.experimental.pallas import tpu as pltpu
from jax.experimental.pallas import tpu_sc as plsc

NC = 2
NS = 16
NW = NC * NS
L = 128
LI = 112


def _sc_mesh():
    return plsc.VectorSubcoreMesh(core_axis_name="c", subcore_axis_name="s",
                                  num_cores=NC, num_subcores=NS)


def _zero_vmem_rows(ref, nrows):
    zeros16 = jnp.zeros((16,), ref.dtype)

    def body(i, _):
        for k in range(8):
            ref[i, pl.ds(k * 16, 16)] = zeros16
        return 0

    lax.fori_loop(0, nrows, body, 0)


def _make_degree_kernel(er, n_pad):
    base = er // NW
    assert base % 8 == 0 and base * NW == er
    stripe = n_pad // NS
    assert stripe % L == 0

    @functools.partial(
        pl.kernel,
        out_type=jax.ShapeDtypeStruct((NC, n_pad), jnp.float32),
        mesh=_sc_mesh(),
        scratch_types=[
            pltpu.VMEM((base, LI), jnp.int32),
            pltpu.VMEM((L,), jnp.float32),
            pltpu.VMEM((L,), jnp.float32),
            pltpu.VMEM_SHARED((n_pad,), jnp.float32),
        ],
    )
    def deg_kernel(dst_hbm, out_hbm, dstbuf, ones_v, zeros_v, acc):
        cid = lax.axis_index("c")
        sid = lax.axis_index("s")
        wid = sid * NC + cid

        for k in range(8):
            ones_v[pl.ds(k * 16, 16)] = jnp.ones((16,), jnp.float32)
            zeros_v[pl.ds(k * 16, 16)] = jnp.zeros((16,), jnp.float32)

        for k in range(stripe // L):
            pltpu.sync_copy(zeros_v, acc.at[pl.ds(sid * stripe + k * L, L)])
        plsc.subcore_barrier()

        pltpu.sync_copy(dst_hbm.at[pl.ds(wid * base, base)], dstbuf)

        def body(r, _):
            pltpu.sync_copy(ones_v.at[pl.ds(0, LI)], acc.at[dstbuf.at[r]],
                            add=True)
            return 0

        lax.fori_loop(0, base, body, 0)

        plsc.subcore_barrier()
        for k in range(stripe // L):
            off = sid * stripe + k * L
            pltpu.sync_copy(acc.at[pl.ds(off, L)], out_hbm.at[cid, pl.ds(off, L)])

    return deg_kernel


def _make_prop_kernel(er, n_pad, d):
    base = er // NW
    assert base % 8 == 0 and base * NW == er
    stripe = n_pad // NS
    assert stripe % L == 0
    KB = 8

    @functools.partial(
        pl.kernel,
        out_type=jax.ShapeDtypeStruct((NC, n_pad, d), jnp.float32),
        mesh=_sc_mesh(),
        scratch_types=[
            pltpu.VMEM((KB, LI), jnp.int32),
            pltpu.VMEM((KB, LI), jnp.int32),
            pltpu.VMEM((LI, d), jnp.float32),
            pltpu.VMEM((LI, d), jnp.float32),
            pltpu.VMEM((LI, d), jnp.float32),
            pltpu.VMEM_SHARED((n_pad, d), jnp.float32),
            pltpu.SemaphoreType.DMA,
        ],
    )
    def prop_kernel(tab_hbm, src_hbm, dst_hbm, out_hbm,
                    srcbuf, dstbuf, rows_a, rows_b, rows_c, acc, sem):
        cid = lax.axis_index("c")
        sid = lax.axis_index("s")
        wid = sid * NC + cid
        bufs = (rows_a, rows_b, rows_c)
        nb = len(bufs)

        _zero_vmem_rows(rows_a, LI)
        off = 0
        while off < stripe:
            cs = min(LI, stripe - off)
            pltpu.sync_copy(rows_a.at[pl.ds(0, cs)],
                            acc.at[pl.ds(sid * stripe + off, cs)])
            off += cs
        plsc.subcore_barrier()

        def chunk(c, _):
            row0 = wid * base + c * KB
            pltpu.sync_copy(src_hbm.at[pl.ds(row0, KB)], srcbuf)
            pltpu.sync_copy(dst_hbm.at[pl.ds(row0, KB)], dstbuf)
            g = [pltpu.async_copy(tab_hbm.at[srcbuf.at[r]], bufs[r % nb], sem)
                 for r in range(nb)]
            for r in range(KB):
                g[r].wait()
                pltpu.sync_copy(bufs[r % nb], acc.at[dstbuf.at[r]], add=True)
                if r + nb < KB:
                    g.append(pltpu.async_copy(tab_hbm.at[srcbuf.at[r + nb]],
                                              bufs[(r + nb) % nb], sem))
            return 0

        lax.fori_loop(0, base // KB, chunk, 0)

        plsc.subcore_barrier()
        pltpu.sync_copy(acc.at[pl.ds(sid * stripe, stripe)],
                        out_hbm.at[cid, pl.ds(sid * stripe, stripe)])

    return prop_kernel


def _prep_tc(degp, x_pad):
    n_pad, d = x_pad.shape

    def body(degp_ref, x_ref, dinv_ref, xs_ref):
        deg = degp_ref[0, :] + degp_ref[1, :] + 1.0
        dinv = lax.rsqrt(jnp.maximum(deg, 1.0))
        dinv2d = jnp.broadcast_to(dinv[:, None], (n_pad, d))
        dinv_ref[...] = dinv2d
        xs_ref[...] = x_ref[...] * dinv2d

    return pl.pallas_call(
        body,
        out_shape=(
            jax.ShapeDtypeStruct((n_pad, d), jnp.float32),
            jax.ShapeDtypeStruct((n_pad, d), jnp.float32),
        ),
    )(degp, x_pad)


def _layer1_tc(partials, xs, dinv2d, w1):
    n_pad, d = xs.shape
    blk = 1024
    grid = n_pad // blk

    def body(p_ref, xs_ref, dinv_ref, w_ref, out_ref):
        agg = (p_ref[0] + p_ref[1] + xs_ref[...]) * dinv_ref[...]
        h = jnp.dot(agg, w_ref[...], preferred_element_type=jnp.float32)
        out_ref[...] = jnp.maximum(h, 0.0) * dinv_ref[...]

    return pl.pallas_call(
        body,
        grid=(grid,),
        in_specs=[
            pl.BlockSpec((NC, blk, d), lambda j: (0, j, 0)),
            pl.BlockSpec((blk, d), lambda j: (j, 0)),
            pl.BlockSpec((blk, d), lambda j: (j, 0)),
            pl.BlockSpec((d, d), lambda j: (0, 0)),
        ],
        out_specs=pl.BlockSpec((blk, d), lambda j: (j, 0)),
        out_shape=jax.ShapeDtypeStruct((n_pad, d), jnp.float32),
    )(partials, xs, dinv2d, w1)


def _final_tc(partials, h1s, dinv2d, w2, img):
    n_pad, d = h1s.shape
    b = img.shape[0]
    blk = 1024
    grid = n_pad // blk

    def body(p_ref, h_ref, dinv_ref, w_ref, img_ref, out_ref):
        agg = (p_ref[0] + p_ref[1] + h_ref[...]) * dinv_ref[...]
        h2 = jnp.dot(agg, w_ref[...], preferred_element_type=jnp.float32)
        out_ref[...] = lax.dot_general(
            img_ref[...], h2,
            dimension_numbers=(((1,), (1,)), ((), ())),
            preferred_element_type=jnp.float32)

    return pl.pallas_call(
        body,
        grid=(grid,),
        in_specs=[
            pl.BlockSpec((NC, blk, d), lambda j: (0, j, 0)),
            pl.BlockSpec((blk, d), lambda j: (j, 0)),
            pl.BlockSpec((blk, d), lambda j: (j, 0)),
            pl.BlockSpec((d, d), lambda j: (0, 0)),
            pl.BlockSpec((b, d), lambda j: (0, 0)),
        ],
        out_specs=pl.BlockSpec((b, blk), lambda j: (0, j)),
        out_shape=jax.ShapeDtypeStruct((b, n_pad), jnp.float32),
    )(partials, h1s, dinv2d, w2, img)


@jax.jit
def _run(x, edge_index, img, w1, w2):
    n, d = x.shape
    e = edge_index.shape[1]
    n_pad = ((n + NS * L - 1) // (NS * L)) * (NS * L)

    base = (-(-e // (NW * LI)) + 7) // 8 * 8
    er = base * NW
    e_pad = er * LI - e
    src_flat = jnp.pad(edge_index[0], (0, e_pad))
    dst_flat = jnp.pad(edge_index[1], (0, e_pad), constant_values=n)
    src2d = src_flat.reshape(er, LI)
    dst2d = dst_flat.reshape(er, LI)
    x_pad = jnp.pad(x, ((0, n_pad - n), (0, 0)))

    degp = _make_degree_kernel(er, n_pad)(dst2d)
    dinv2d, xs = _prep_tc(degp, x_pad)

    prop = _make_prop_kernel(er, n_pad, d)
    p1 = prop(xs, src2d, dst2d)
    h1s = _layer1_tc(p1, xs, dinv2d, w1)
    p2 = prop(h1s, src2d, dst2d)
    scores_pad = _final_tc(p2, h1s, dinv2d, w2, img)
    return scores_pad[:, :n]


def kernel(x, edge_index, img, W1, W2):
    return _run(x, edge_index, img, W1, W2)


# speedup vs baseline: 1.9307x; 1.9307x over previous
import functools

import jax
import jax.numpy as jnp
from jax import lax
from jax.experimental import pallas as pl
from jax.experimental.pallas import tpu as pltpu
from jax.experimental.pallas import tpu_sc as plsc

NC = 2
NS = 16
NW = NC * NS
L = 128
LI = 128


def _sc_mesh():
    return plsc.VectorSubcoreMesh(core_axis_name="c", subcore_axis_name="s",
                                  num_cores=NC, num_subcores=NS)


def _zero_vmem_rows(ref, nrows):
    zeros16 = jnp.zeros((16,), ref.dtype)

    def body(i, _):
        for k in range(8):
            ref[i, pl.ds(k * 16, 16)] = zeros16
        return 0

    lax.fori_loop(0, nrows, body, 0)


def _make_degree_kernel(er, n_pad):
    base = er // NW
    assert base % 8 == 0 and base * NW == er
    stripe = n_pad // NS
    assert stripe % L == 0

    @functools.partial(
        pl.kernel,
        out_type=jax.ShapeDtypeStruct((NC, n_pad), jnp.float32),
        mesh=_sc_mesh(),
        scratch_types=[
            pltpu.VMEM((base, LI), jnp.int32),
            pltpu.VMEM((L,), jnp.float32),
            pltpu.VMEM((L,), jnp.float32),
            pltpu.VMEM_SHARED((n_pad,), jnp.float32),
        ],
    )
    def deg_kernel(dst_hbm, out_hbm, dstbuf, ones_v, zeros_v, acc):
        cid = lax.axis_index("c")
        sid = lax.axis_index("s")
        wid = sid * NC + cid

        for k in range(8):
            ones_v[pl.ds(k * 16, 16)] = jnp.ones((16,), jnp.float32)
            zeros_v[pl.ds(k * 16, 16)] = jnp.zeros((16,), jnp.float32)

        for k in range(stripe // L):
            pltpu.sync_copy(zeros_v, acc.at[pl.ds(sid * stripe + k * L, L)])
        plsc.subcore_barrier()

        pltpu.sync_copy(dst_hbm.at[pl.ds(wid * base, base)], dstbuf)

        def body(r, _):
            pltpu.sync_copy(ones_v.at[pl.ds(0, LI)], acc.at[dstbuf.at[r]],
                            add=True)
            return 0

        lax.fori_loop(0, base, body, 0)

        plsc.subcore_barrier()
        for k in range(stripe // L):
            off = sid * stripe + k * L
            pltpu.sync_copy(acc.at[pl.ds(off, L)], out_hbm.at[cid, pl.ds(off, L)])

    return deg_kernel


def _make_prop_kernel(er, n_pad, d):
    base = er // NW
    assert base % 8 == 0 and base * NW == er
    stripe = n_pad // NS
    assert stripe % L == 0
    KB = 8

    @functools.partial(
        pl.kernel,
        out_type=jax.ShapeDtypeStruct((NC, n_pad, d), jnp.float32),
        mesh=_sc_mesh(),
        scratch_types=[
            pltpu.VMEM((KB, LI), jnp.int32),
            pltpu.VMEM((KB, LI), jnp.int32),
            pltpu.VMEM((LI, d), jnp.float32),
            pltpu.VMEM((LI, d), jnp.float32),
            pltpu.VMEM_SHARED((n_pad, d), jnp.float32),
            pltpu.SemaphoreType.DMA,
        ],
    )
    def prop_kernel(tab_hbm, src_hbm, dst_hbm, out_hbm,
                    srcbuf, dstbuf, rows_a, rows_b, acc, sem):
        cid = lax.axis_index("c")
        sid = lax.axis_index("s")
        wid = sid * NC + cid
        bufs = (rows_a, rows_b)
        nb = len(bufs)
        H = LI // 2

        def fire_halves(r, buf):
            return (
                pltpu.async_copy(tab_hbm.at[srcbuf.at[r, pl.ds(0, H)]],
                                 buf.at[pl.ds(0, H)], sem),
                pltpu.async_copy(tab_hbm.at[srcbuf.at[r, pl.ds(H, H)]],
                                 buf.at[pl.ds(H, H)], sem),
            )

        _zero_vmem_rows(rows_a, LI)
        off = 0
        while off < stripe:
            cs = min(LI, stripe - off)
            pltpu.sync_copy(rows_a.at[pl.ds(0, cs)],
                            acc.at[pl.ds(sid * stripe + off, cs)])
            off += cs
        plsc.subcore_barrier()

        def chunk(c, _):
            row0 = wid * base + c * KB
            pltpu.sync_copy(src_hbm.at[pl.ds(row0, KB)], srcbuf)
            pltpu.sync_copy(dst_hbm.at[pl.ds(row0, KB)], dstbuf)
            g = [fire_halves(r, bufs[r % nb]) for r in range(nb)]
            for r in range(KB):
                for desc in g[r]:
                    desc.wait()
                pltpu.sync_copy(bufs[r % nb], acc.at[dstbuf.at[r]], add=True)
                if r + nb < KB:
                    g.append(fire_halves(r + nb, bufs[(r + nb) % nb]))
            return 0

        lax.fori_loop(0, base // KB, chunk, 0)

        plsc.subcore_barrier()
        pltpu.sync_copy(acc.at[pl.ds(sid * stripe, stripe)],
                        out_hbm.at[cid, pl.ds(sid * stripe, stripe)])

    return prop_kernel


def _prep_tc(degp, x_pad):
    n_pad, d = x_pad.shape

    def body(degp_ref, x_ref, dinv_ref, xs_ref):
        deg = degp_ref[0, :] + degp_ref[1, :] + 1.0
        dinv = lax.rsqrt(jnp.maximum(deg, 1.0))
        dinv2d = jnp.broadcast_to(dinv[:, None], (n_pad, d))
        dinv_ref[...] = dinv2d
        xs_ref[...] = x_ref[...] * dinv2d

    return pl.pallas_call(
        body,
        out_shape=(
            jax.ShapeDtypeStruct((n_pad, d), jnp.float32),
            jax.ShapeDtypeStruct((n_pad, d), jnp.float32),
        ),
    )(degp, x_pad)


def _layer1_tc(partials, xs, dinv2d, w1):
    n_pad, d = xs.shape
    blk = 1024
    grid = n_pad // blk

    def body(p_ref, xs_ref, dinv_ref, w_ref, out_ref):
        agg = (p_ref[0] + p_ref[1] + xs_ref[...]) * dinv_ref[...]
        h = jnp.dot(agg, w_ref[...], preferred_element_type=jnp.float32)
        out_ref[...] = jnp.maximum(h, 0.0) * dinv_ref[...]

    return pl.pallas_call(
        body,
        grid=(grid,),
        in_specs=[
            pl.BlockSpec((NC, blk, d), lambda j: (0, j, 0)),
            pl.BlockSpec((blk, d), lambda j: (j, 0)),
            pl.BlockSpec((blk, d), lambda j: (j, 0)),
            pl.BlockSpec((d, d), lambda j: (0, 0)),
        ],
        out_specs=pl.BlockSpec((blk, d), lambda j: (j, 0)),
        out_shape=jax.ShapeDtypeStruct((n_pad, d), jnp.float32),
    )(partials, xs, dinv2d, w1)


def _final_tc(partials, h1s, dinv2d, w2, img):
    n_pad, d = h1s.shape
    b = img.shape[0]
    blk = 1024
    grid = n_pad // blk

    def body(p_ref, h_ref, dinv_ref, w_ref, img_ref, out_ref):
        agg = (p_ref[0] + p_ref[1] + h_ref[...]) * dinv_ref[...]
        h2 = jnp.dot(agg, w_ref[...], preferred_element_type=jnp.float32)
        out_ref[...] = lax.dot_general(
            img_ref[...], h2,
            dimension_numbers=(((1,), (1,)), ((), ())),
            preferred_element_type=jnp.float32)

    return pl.pallas_call(
        body,
        grid=(grid,),
        in_specs=[
            pl.BlockSpec((NC, blk, d), lambda j: (0, j, 0)),
            pl.BlockSpec((blk, d), lambda j: (j, 0)),
            pl.BlockSpec((blk, d), lambda j: (j, 0)),
            pl.BlockSpec((d, d), lambda j: (0, 0)),
            pl.BlockSpec((b, d), lambda j: (0, 0)),
        ],
        out_specs=pl.BlockSpec((b, blk), lambda j: (0, j)),
        out_shape=jax.ShapeDtypeStruct((b, n_pad), jnp.float32),
    )(partials, h1s, dinv2d, w2, img)


@jax.jit
def _run(x, edge_index, img, w1, w2):
    n, d = x.shape
    e = edge_index.shape[1]
    n_pad = ((n + NS * L - 1) // (NS * L)) * (NS * L)

    base = (-(-e // (NW * LI)) + 7) // 8 * 8
    er = base * NW
    e_pad = er * LI - e
    src_flat = jnp.pad(edge_index[0], (0, e_pad))
    dst_flat = jnp.pad(edge_index[1], (0, e_pad), constant_values=n)
    src2d = src_flat.reshape(er, LI)
    dst2d = dst_flat.reshape(er, LI)
    x_pad = jnp.pad(x, ((0, n_pad - n), (0, 0)))

    degp = _make_degree_kernel(er, n_pad)(dst2d)
    dinv2d, xs = _prep_tc(degp, x_pad)

    prop = _make_prop_kernel(er, n_pad, d)
    p1 = prop(xs, src2d, dst2d)
    h1s = _layer1_tc(p1, xs, dinv2d, w1)
    p2 = prop(h1s, src2d, dst2d)
    scores_pad = _final_tc(p2, h1s, dinv2d, w2, img)
    return scores_pad[:, :n]


def kernel(x, edge_index, img, W1, W2):
    return _run(x, edge_index, img, W1, W2)


# speedup vs baseline: 2.0502x; 1.0619x over previous
import functools

import jax
import jax.numpy as jnp
from jax import lax
from jax.experimental import pallas as pl
from jax.experimental.pallas import tpu as pltpu
from jax.experimental.pallas import tpu_sc as plsc

NC = 2
NS = 16
NW = NC * NS
L = 128
LI = 128


def _sc_mesh():
    return plsc.VectorSubcoreMesh(core_axis_name="c", subcore_axis_name="s",
                                  num_cores=NC, num_subcores=NS)


def _zero_vmem_rows(ref, nrows):
    zeros16 = jnp.zeros((16,), ref.dtype)

    def body(i, _):
        for k in range(8):
            ref[i, pl.ds(k * 16, 16)] = zeros16
        return 0

    lax.fori_loop(0, nrows, body, 0)


def _make_degree_kernel(er, n_pad):
    base = er // NW
    assert base % 8 == 0 and base * NW == er
    stripe = n_pad // NS
    assert stripe % L == 0

    @functools.partial(
        pl.kernel,
        out_type=jax.ShapeDtypeStruct((NC, n_pad), jnp.float32),
        mesh=_sc_mesh(),
        scratch_types=[
            pltpu.VMEM((base, LI), jnp.int32),
            pltpu.VMEM((L,), jnp.float32),
            pltpu.VMEM((L,), jnp.float32),
            pltpu.VMEM_SHARED((n_pad,), jnp.float32),
        ],
    )
    def deg_kernel(dst_hbm, out_hbm, dstbuf, ones_v, zeros_v, acc):
        cid = lax.axis_index("c")
        sid = lax.axis_index("s")
        wid = sid * NC + cid

        for k in range(8):
            ones_v[pl.ds(k * 16, 16)] = jnp.ones((16,), jnp.float32)
            zeros_v[pl.ds(k * 16, 16)] = jnp.zeros((16,), jnp.float32)

        for k in range(stripe // L):
            pltpu.sync_copy(zeros_v, acc.at[pl.ds(sid * stripe + k * L, L)])
        plsc.subcore_barrier()

        pltpu.sync_copy(dst_hbm.at[pl.ds(wid * base, base)], dstbuf)

        def body(r, _):
            pltpu.sync_copy(ones_v.at[pl.ds(0, LI)], acc.at[dstbuf.at[r]],
                            add=True)
            return 0

        lax.fori_loop(0, base, body, 0)

        plsc.subcore_barrier()
        for k in range(stripe // L):
            off = sid * stripe + k * L
            pltpu.sync_copy(acc.at[pl.ds(off, L)], out_hbm.at[cid, pl.ds(off, L)])

    return deg_kernel


def _make_prop_kernel(er, n_pad, d, skew=0):
    base = er // NW
    assert base % 8 == 0 and base * NW == er
    b0, b1 = base + skew, base - skew
    assert b0 % 8 == 0 and b1 % 8 == 0 and b0 >= 0 and b1 >= 0
    stripe = n_pad // NS
    assert stripe % L == 0
    KB = 8

    @functools.partial(
        pl.kernel,
        out_type=jax.ShapeDtypeStruct((NC, n_pad, d), jnp.float32),
        mesh=_sc_mesh(),
        scratch_types=[
            pltpu.VMEM((KB, LI), jnp.int32),
            pltpu.VMEM((KB, LI), jnp.int32),
            pltpu.VMEM((LI, d), jnp.float32),
            pltpu.VMEM((LI, d), jnp.float32),
            pltpu.VMEM_SHARED((n_pad, d), jnp.float32),
            pltpu.SemaphoreType.DMA,
        ],
    )
    def prop_kernel(tab_hbm, src_hbm, dst_hbm, out_hbm,
                    srcbuf, dstbuf, rows_a, rows_b, acc, sem):
        cid = lax.axis_index("c")
        sid = lax.axis_index("s")
        wid = sid * NC + cid
        bufs = (rows_a, rows_b)
        nb = len(bufs)
        H = LI // 2

        def fire_halves(r, buf):
            return (
                pltpu.async_copy(tab_hbm.at[srcbuf.at[r, pl.ds(0, H)]],
                                 buf.at[pl.ds(0, H)], sem),
                pltpu.async_copy(tab_hbm.at[srcbuf.at[r, pl.ds(H, H)]],
                                 buf.at[pl.ds(H, H)], sem),
            )

        _zero_vmem_rows(rows_a, LI)
        off = 0
        while off < stripe:
            cs = min(LI, stripe - off)
            pltpu.sync_copy(rows_a.at[pl.ds(0, cs)],
                            acc.at[pl.ds(sid * stripe + off, cs)])
            off += cs
        plsc.subcore_barrier()

        my_rows = jnp.where(cid == 0, b0, b1)
        my_row0 = jnp.where(cid == 0, sid * b0, NS * b0 + sid * b1)

        def chunk(c, _):
            row0 = my_row0 + c * KB
            pltpu.sync_copy(src_hbm.at[pl.ds(row0, KB)], srcbuf)
            pltpu.sync_copy(dst_hbm.at[pl.ds(row0, KB)], dstbuf)
            g = [fire_halves(r, bufs[r % nb]) for r in range(nb)]
            for r in range(KB):
                for desc in g[r]:
                    desc.wait()
                pltpu.sync_copy(bufs[r % nb], acc.at[dstbuf.at[r]], add=True)
                if r + nb < KB:
                    g.append(fire_halves(r + nb, bufs[(r + nb) % nb]))
            return 0

        lax.fori_loop(0, my_rows // KB, chunk, 0)

        plsc.subcore_barrier()
        pltpu.sync_copy(acc.at[pl.ds(sid * stripe, stripe)],
                        out_hbm.at[cid, pl.ds(sid * stripe, stripe)])

    return prop_kernel


def _prep_tc(degp, x_pad):
    n_pad, d = x_pad.shape

    def body(degp_ref, x_ref, dinv_ref, xs_ref):
        deg = degp_ref[0, :] + degp_ref[1, :] + 1.0
        dinv = lax.rsqrt(jnp.maximum(deg, 1.0))
        dinv2d = jnp.broadcast_to(dinv[:, None], (n_pad, d))
        dinv_ref[...] = dinv2d
        xs_ref[...] = x_ref[...] * dinv2d

    return pl.pallas_call(
        body,
        out_shape=(
            jax.ShapeDtypeStruct((n_pad, d), jnp.float32),
            jax.ShapeDtypeStruct((n_pad, d), jnp.float32),
        ),
    )(degp, x_pad)


def _layer1_tc(partials, xs, dinv2d, w1):
    n_pad, d = xs.shape
    blk = 1024
    grid = n_pad // blk

    def body(p_ref, xs_ref, dinv_ref, w_ref, out_ref):
        agg = (p_ref[0] + p_ref[1] + xs_ref[...]) * dinv_ref[...]
        h = jnp.dot(agg, w_ref[...], preferred_element_type=jnp.float32)
        out_ref[...] = jnp.maximum(h, 0.0) * dinv_ref[...]

    return pl.pallas_call(
        body,
        grid=(grid,),
        in_specs=[
            pl.BlockSpec((NC, blk, d), lambda j: (0, j, 0)),
            pl.BlockSpec((blk, d), lambda j: (j, 0)),
            pl.BlockSpec((blk, d), lambda j: (j, 0)),
            pl.BlockSpec((d, d), lambda j: (0, 0)),
        ],
        out_specs=pl.BlockSpec((blk, d), lambda j: (j, 0)),
        out_shape=jax.ShapeDtypeStruct((n_pad, d), jnp.float32),
    )(partials, xs, dinv2d, w1)


def _final_tc(partials, h1s, dinv2d, w2, img):
    n_pad, d = h1s.shape
    b = img.shape[0]
    blk = 1024
    grid = n_pad // blk

    def body(p_ref, h_ref, dinv_ref, w_ref, img_ref, out_ref):
        agg = (p_ref[0] + p_ref[1] + h_ref[...]) * dinv_ref[...]
        h2 = jnp.dot(agg, w_ref[...], preferred_element_type=jnp.float32)
        out_ref[...] = lax.dot_general(
            img_ref[...], h2,
            dimension_numbers=(((1,), (1,)), ((), ())),
            preferred_element_type=jnp.float32)

    return pl.pallas_call(
        body,
        grid=(grid,),
        in_specs=[
            pl.BlockSpec((NC, blk, d), lambda j: (0, j, 0)),
            pl.BlockSpec((blk, d), lambda j: (j, 0)),
            pl.BlockSpec((blk, d), lambda j: (j, 0)),
            pl.BlockSpec((d, d), lambda j: (0, 0)),
            pl.BlockSpec((b, d), lambda j: (0, 0)),
        ],
        out_specs=pl.BlockSpec((b, blk), lambda j: (0, j)),
        out_shape=jax.ShapeDtypeStruct((b, n_pad), jnp.float32),
    )(partials, h1s, dinv2d, w2, img)


@jax.jit
def _run(x, edge_index, img, w1, w2):
    n, d = x.shape
    e = edge_index.shape[1]
    n_pad = ((n + NS * L - 1) // (NS * L)) * (NS * L)

    base = (-(-e // (NW * LI)) + 7) // 8 * 8
    er = base * NW
    e_pad = er * LI - e
    src_flat = jnp.pad(edge_index[0], (0, e_pad))
    dst_flat = jnp.pad(edge_index[1], (0, e_pad), constant_values=n)
    src2d = src_flat.reshape(er, LI)
    dst2d = dst_flat.reshape(er, LI)
    x_pad = jnp.pad(x, ((0, n_pad - n), (0, 0)))

    degp = _make_degree_kernel(er, n_pad)(dst2d)
    dinv2d, xs = _prep_tc(degp, x_pad)

    prop = _make_prop_kernel(er, n_pad, d, skew=40)
    p1 = prop(xs, src2d, dst2d)
    h1s = _layer1_tc(p1, xs, dinv2d, w1)
    p2 = prop(h1s, src2d, dst2d)
    scores_pad = _final_tc(p2, h1s, dinv2d, w2, img)
    return scores_pad[:, :n]


def kernel(x, edge_index, img, W1, W2):
    return _run(x, edge_index, img, W1, W2)


# speedup vs baseline: 2.1553x; 1.0513x over previous
import functools

import jax
import jax.numpy as jnp
from jax import lax
from jax.experimental import pallas as pl
from jax.experimental.pallas import tpu as pltpu
from jax.experimental.pallas import tpu_sc as plsc

NC = 2
NS = 16
NW = NC * NS
L = 128
LI = 128


def _sc_mesh():
    return plsc.VectorSubcoreMesh(core_axis_name="c", subcore_axis_name="s",
                                  num_cores=NC, num_subcores=NS)


def _zero_vmem_rows(ref, nrows):
    zeros16 = jnp.zeros((16,), ref.dtype)

    def body(i, _):
        for k in range(8):
            ref[i, pl.ds(k * 16, 16)] = zeros16
        return 0

    lax.fori_loop(0, nrows, body, 0)


def _make_degree_kernel(er, n_pad):
    base = er // NW
    assert base % 8 == 0 and base * NW == er
    stripe = n_pad // NS
    assert stripe % L == 0

    @functools.partial(
        pl.kernel,
        out_type=jax.ShapeDtypeStruct((NC, n_pad), jnp.float32),
        mesh=_sc_mesh(),
        scratch_types=[
            pltpu.VMEM((base, LI), jnp.int32),
            pltpu.VMEM((L,), jnp.float32),
            pltpu.VMEM((L,), jnp.float32),
            pltpu.VMEM_SHARED((n_pad,), jnp.float32),
        ],
    )
    def deg_kernel(dst_hbm, out_hbm, dstbuf, ones_v, zeros_v, acc):
        cid = lax.axis_index("c")
        sid = lax.axis_index("s")
        wid = sid * NC + cid

        for k in range(8):
            ones_v[pl.ds(k * 16, 16)] = jnp.ones((16,), jnp.float32)
            zeros_v[pl.ds(k * 16, 16)] = jnp.zeros((16,), jnp.float32)

        for k in range(stripe // L):
            pltpu.sync_copy(zeros_v, acc.at[pl.ds(sid * stripe + k * L, L)])
        plsc.subcore_barrier()

        pltpu.sync_copy(dst_hbm.at[pl.ds(wid * base, base)], dstbuf)

        def body(r, _):
            pltpu.sync_copy(ones_v.at[pl.ds(0, LI)], acc.at[dstbuf.at[r]],
                            add=True)
            return 0

        lax.fori_loop(0, base, body, 0)

        plsc.subcore_barrier()
        for k in range(stripe // L):
            off = sid * stripe + k * L
            pltpu.sync_copy(acc.at[pl.ds(off, L)], out_hbm.at[cid, pl.ds(off, L)])

    return deg_kernel


def _make_prop_kernel(er, n_pad, d, skew=0):
    base = er // NW
    assert base % 8 == 0 and base * NW == er
    b0, b1 = base + skew, base - skew
    assert b0 % 8 == 0 and b1 % 8 == 0 and b0 >= 0 and b1 >= 0
    stripe = n_pad // NS
    assert stripe % L == 0
    KB = 8

    @functools.partial(
        pl.kernel,
        out_type=jax.ShapeDtypeStruct((NC, n_pad, d), jnp.float32),
        mesh=_sc_mesh(),
        scratch_types=[
            pltpu.VMEM((KB, LI), jnp.int32),
            pltpu.VMEM((KB, LI), jnp.int32),
            pltpu.VMEM((LI, d), jnp.float32),
            pltpu.VMEM((LI, d), jnp.float32),
            pltpu.VMEM_SHARED((n_pad, d), jnp.float32),
            pltpu.SemaphoreType.DMA,
        ],
    )
    def prop_kernel(tab_hbm, src_hbm, dst_hbm, out_hbm,
                    srcbuf, dstbuf, rows_a, rows_b, acc, sem):
        cid = lax.axis_index("c")
        sid = lax.axis_index("s")
        wid = sid * NC + cid
        bufs = (rows_a, rows_b)
        nb = len(bufs)
        H = LI // 2

        def fire_halves(r, buf):
            return (
                pltpu.async_copy(tab_hbm.at[srcbuf.at[r, pl.ds(0, H)]],
                                 buf.at[pl.ds(0, H)], sem),
                pltpu.async_copy(tab_hbm.at[srcbuf.at[r, pl.ds(H, H)]],
                                 buf.at[pl.ds(H, H)], sem),
            )

        _zero_vmem_rows(rows_a, LI)
        off = 0
        while off < stripe:
            cs = min(LI, stripe - off)
            pltpu.sync_copy(rows_a.at[pl.ds(0, cs)],
                            acc.at[pl.ds(sid * stripe + off, cs)])
            off += cs
        plsc.subcore_barrier()

        my_rows = jnp.where(cid == 0, b0, b1)
        my_row0 = jnp.where(cid == 0, sid * b0, NS * b0 + sid * b1)

        def chunk(c, _):
            row0 = my_row0 + c * KB
            pltpu.sync_copy(src_hbm.at[pl.ds(row0, KB)], srcbuf)
            pltpu.sync_copy(dst_hbm.at[pl.ds(row0, KB)], dstbuf)
            g = [fire_halves(r, bufs[r % nb]) for r in range(nb)]
            for r in range(KB):
                for desc in g[r]:
                    desc.wait()
                pltpu.sync_copy(bufs[r % nb], acc.at[dstbuf.at[r]], add=True)
                if r + nb < KB:
                    g.append(fire_halves(r + nb, bufs[(r + nb) % nb]))
            return 0

        lax.fori_loop(0, my_rows // KB, chunk, 0)

        plsc.subcore_barrier()
        pltpu.sync_copy(acc.at[pl.ds(sid * stripe, stripe)],
                        out_hbm.at[cid, pl.ds(sid * stripe, stripe)])

    return prop_kernel


def _prep_tc(degp, x_pad):
    n_pad, d = x_pad.shape

    def body(degp_ref, x_ref, dinv_ref, xs_ref):
        deg = degp_ref[0, :] + degp_ref[1, :] + 1.0
        dinv = lax.rsqrt(jnp.maximum(deg, 1.0))
        dinv2d = jnp.broadcast_to(dinv[:, None], (n_pad, d))
        dinv_ref[...] = dinv2d
        xs_ref[...] = x_ref[...] * dinv2d

    return pl.pallas_call(
        body,
        out_shape=(
            jax.ShapeDtypeStruct((n_pad, d), jnp.float32),
            jax.ShapeDtypeStruct((n_pad, d), jnp.float32),
        ),
    )(degp, x_pad)


def _layer1_tc(partials, xs, dinv2d, w1):
    n_pad, d = xs.shape
    blk = 1024
    grid = n_pad // blk

    def body(p_ref, xs_ref, dinv_ref, w_ref, out_ref):
        agg = (p_ref[0] + p_ref[1] + xs_ref[...]) * dinv_ref[...]
        h = jnp.dot(agg, w_ref[...], preferred_element_type=jnp.float32)
        out_ref[...] = jnp.maximum(h, 0.0) * dinv_ref[...]

    return pl.pallas_call(
        body,
        grid=(grid,),
        in_specs=[
            pl.BlockSpec((NC, blk, d), lambda j: (0, j, 0)),
            pl.BlockSpec((blk, d), lambda j: (j, 0)),
            pl.BlockSpec((blk, d), lambda j: (j, 0)),
            pl.BlockSpec((d, d), lambda j: (0, 0)),
        ],
        out_specs=pl.BlockSpec((blk, d), lambda j: (j, 0)),
        out_shape=jax.ShapeDtypeStruct((n_pad, d), jnp.float32),
    )(partials, xs, dinv2d, w1)


def _final_tc(partials, h1s, dinv2d, w2, img):
    n_pad, d = h1s.shape
    b = img.shape[0]
    blk = 1024
    grid = n_pad // blk

    def body(p_ref, h_ref, dinv_ref, w_ref, img_ref, out_ref):
        agg = (p_ref[0] + p_ref[1] + h_ref[...]) * dinv_ref[...]
        h2 = jnp.dot(agg, w_ref[...], preferred_element_type=jnp.float32)
        out_ref[...] = lax.dot_general(
            img_ref[...], h2,
            dimension_numbers=(((1,), (1,)), ((), ())),
            preferred_element_type=jnp.float32)

    return pl.pallas_call(
        body,
        grid=(grid,),
        in_specs=[
            pl.BlockSpec((NC, blk, d), lambda j: (0, j, 0)),
            pl.BlockSpec((blk, d), lambda j: (j, 0)),
            pl.BlockSpec((blk, d), lambda j: (j, 0)),
            pl.BlockSpec((d, d), lambda j: (0, 0)),
            pl.BlockSpec((b, d), lambda j: (0, 0)),
        ],
        out_specs=pl.BlockSpec((b, blk), lambda j: (0, j)),
        out_shape=jax.ShapeDtypeStruct((b, n_pad), jnp.float32),
    )(partials, h1s, dinv2d, w2, img)


@jax.jit
def _run(x, edge_index, img, w1, w2):
    n, d = x.shape
    e = edge_index.shape[1]
    n_pad = ((n + NS * L - 1) // (NS * L)) * (NS * L)

    base = (-(-e // (NW * LI)) + 7) // 8 * 8
    er = base * NW
    e_pad = er * LI - e
    src_flat = jnp.pad(edge_index[0], (0, e_pad))
    dst_flat = jnp.pad(edge_index[1], (0, e_pad), constant_values=n)
    src2d = src_flat.reshape(er, LI)
    dst2d = dst_flat.reshape(er, LI)
    x_pad = jnp.pad(x, ((0, n_pad - n), (0, 0)))

    degp = _make_degree_kernel(er, n_pad)(dst2d)
    dinv2d, xs = _prep_tc(degp, x_pad)

    prop = _make_prop_kernel(er, n_pad, d, skew=56)
    p1 = prop(xs, src2d, dst2d)
    h1s = _layer1_tc(p1, xs, dinv2d, w1)
    p2 = prop(h1s, src2d, dst2d)
    scores_pad = _final_tc(p2, h1s, dinv2d, w2, img)
    return scores_pad[:, :n]


def kernel(x, edge_index, img, W1, W2):
    return _run(x, edge_index, img, W1, W2)


# speedup vs baseline: 2.4191x; 1.1224x over previous
import functools

import jax
import jax.numpy as jnp
from jax import lax
from jax.experimental import pallas as pl
from jax.experimental.pallas import tpu as pltpu
from jax.experimental.pallas import tpu_sc as plsc

NC = 2
NS = 16
NW = NC * NS
L = 128
LI = 128


def _sc_mesh():
    return plsc.VectorSubcoreMesh(core_axis_name="c", subcore_axis_name="s",
                                  num_cores=NC, num_subcores=NS)


def _zero_vmem_rows(ref, nrows):
    zeros16 = jnp.zeros((16,), ref.dtype)

    def body(i, _):
        for k in range(8):
            ref[i, pl.ds(k * 16, 16)] = zeros16
        return 0

    lax.fori_loop(0, nrows, body, 0)


def _make_degree_kernel(er, n_pad):
    base = er // NW
    assert base % 8 == 0 and base * NW == er
    stripe = n_pad // NS
    assert stripe % L == 0

    @functools.partial(
        pl.kernel,
        out_type=jax.ShapeDtypeStruct((NC, n_pad), jnp.float32),
        mesh=_sc_mesh(),
        scratch_types=[
            pltpu.VMEM((base, LI), jnp.int32),
            pltpu.VMEM((L,), jnp.float32),
            pltpu.VMEM((L,), jnp.float32),
            pltpu.VMEM_SHARED((n_pad,), jnp.float32),
        ],
    )
    def deg_kernel(dst_hbm, out_hbm, dstbuf, ones_v, zeros_v, acc):
        cid = lax.axis_index("c")
        sid = lax.axis_index("s")
        wid = sid * NC + cid

        for k in range(8):
            ones_v[pl.ds(k * 16, 16)] = jnp.ones((16,), jnp.float32)
            zeros_v[pl.ds(k * 16, 16)] = jnp.zeros((16,), jnp.float32)

        for k in range(stripe // L):
            pltpu.sync_copy(zeros_v, acc.at[pl.ds(sid * stripe + k * L, L)])
        plsc.subcore_barrier()

        pltpu.sync_copy(dst_hbm.at[pl.ds(wid * base, base)], dstbuf)

        def body(r, _):
            pltpu.sync_copy(ones_v.at[pl.ds(0, LI)], acc.at[dstbuf.at[r]],
                            add=True)
            return 0

        lax.fori_loop(0, base, body, 0)

        plsc.subcore_barrier()
        for k in range(stripe // L):
            off = sid * stripe + k * L
            pltpu.sync_copy(acc.at[pl.ds(off, L)], out_hbm.at[cid, pl.ds(off, L)])

    return deg_kernel


def _make_prop_kernel(er, n_pad, d, skew=0):
    base = er // NW
    assert base % 8 == 0 and base * NW == er
    b0, b1 = base + skew, base - skew
    assert b0 % 8 == 0 and b1 % 8 == 0 and b0 >= 0 and b1 >= 0
    stripe = n_pad // NS
    assert stripe % L == 0
    KB = 8

    @functools.partial(
        pl.kernel,
        out_type=jax.ShapeDtypeStruct((NC, n_pad, d), jnp.float32),
        mesh=_sc_mesh(),
        scratch_types=[
            pltpu.VMEM((KB, LI), jnp.int32),
            pltpu.VMEM((KB, LI), jnp.int32),
            pltpu.VMEM((LI, d), jnp.float32),
            pltpu.VMEM((LI, d), jnp.float32),
            pltpu.VMEM_SHARED((n_pad, d), jnp.float32),
            pltpu.SemaphoreType.DMA,
        ],
    )
    def prop_kernel(tab_hbm, src_hbm, dst_hbm, out_hbm,
                    srcbuf, dstbuf, rows_a, rows_b, acc, sem):
        cid = lax.axis_index("c")
        sid = lax.axis_index("s")
        wid = sid * NC + cid
        bufs = (rows_a, rows_b)
        nb = len(bufs)
        H = LI // 2

        def fire_halves(r, buf):
            return (
                pltpu.async_copy(tab_hbm.at[srcbuf.at[r, pl.ds(0, H)]],
                                 buf.at[pl.ds(0, H)], sem),
                pltpu.async_copy(tab_hbm.at[srcbuf.at[r, pl.ds(H, H)]],
                                 buf.at[pl.ds(H, H)], sem),
            )

        _zero_vmem_rows(rows_a, LI)
        off = 0
        while off < stripe:
            cs = min(LI, stripe - off)
            pltpu.sync_copy(rows_a.at[pl.ds(0, cs)],
                            acc.at[pl.ds(sid * stripe + off, cs)])
            off += cs
        plsc.subcore_barrier()

        my_rows = jnp.where(cid == 0, b0, b1)
        my_row0 = jnp.where(cid == 0, sid * b0, NS * b0 + sid * b1)

        def chunk(c, _):
            row0 = my_row0 + c * KB
            pltpu.sync_copy(src_hbm.at[pl.ds(row0, KB)], srcbuf)
            pltpu.sync_copy(dst_hbm.at[pl.ds(row0, KB)], dstbuf)
            g = [fire_halves(r, bufs[r % nb]) for r in range(nb)]
            for r in range(KB):
                for desc in g[r]:
                    desc.wait()
                pltpu.sync_copy(bufs[r % nb], acc.at[dstbuf.at[r]], add=True)
                if r + nb < KB:
                    g.append(fire_halves(r + nb, bufs[(r + nb) % nb]))
            return 0

        lax.fori_loop(0, my_rows // KB, chunk, 0)

        plsc.subcore_barrier()
        pltpu.sync_copy(acc.at[pl.ds(sid * stripe, stripe)],
                        out_hbm.at[cid, pl.ds(sid * stripe, stripe)])

    return prop_kernel


def _prep_tc(degp, x_pad):
    n_pad, d = x_pad.shape

    def body(degp_ref, x_ref, dinv_ref, xs_ref):
        deg = degp_ref[0, :] + degp_ref[1, :] + 1.0
        dinv = lax.rsqrt(jnp.maximum(deg, 1.0))
        dinv2d = jnp.broadcast_to(dinv[:, None], (n_pad, d))
        dinv_ref[...] = dinv2d
        xs_ref[...] = x_ref[...] * dinv2d

    return pl.pallas_call(
        body,
        out_shape=(
            jax.ShapeDtypeStruct((n_pad, d), jnp.float32),
            jax.ShapeDtypeStruct((n_pad, d), jnp.float32),
        ),
    )(degp, x_pad)


def _layer1_tc(partials, xs, dinv2d, w1):
    n_pad, d = xs.shape
    blk = 1024
    grid = n_pad // blk

    def body(p_ref, xs_ref, dinv_ref, w_ref, out_ref):
        agg = (p_ref[0] + p_ref[1] + xs_ref[...]) * dinv_ref[...]
        h = jnp.dot(agg, w_ref[...], preferred_element_type=jnp.float32)
        out_ref[...] = jnp.maximum(h, 0.0) * dinv_ref[...]

    return pl.pallas_call(
        body,
        grid=(grid,),
        in_specs=[
            pl.BlockSpec((NC, blk, d), lambda j: (0, j, 0)),
            pl.BlockSpec((blk, d), lambda j: (j, 0)),
            pl.BlockSpec((blk, d), lambda j: (j, 0)),
            pl.BlockSpec((d, d), lambda j: (0, 0)),
        ],
        out_specs=pl.BlockSpec((blk, d), lambda j: (j, 0)),
        out_shape=jax.ShapeDtypeStruct((n_pad, d), jnp.float32),
    )(partials, xs, dinv2d, w1)


def _final_tc(partials, h1s, dinv2d, w2, img):
    n_pad, d = h1s.shape
    b = img.shape[0]
    blk = 1024
    grid = n_pad // blk

    def body(p_ref, h_ref, dinv_ref, w_ref, img_ref, out_ref):
        agg = (p_ref[0] + p_ref[1] + h_ref[...]) * dinv_ref[...]
        h2 = jnp.dot(agg, w_ref[...], preferred_element_type=jnp.float32)
        out_ref[...] = lax.dot_general(
            img_ref[...], h2,
            dimension_numbers=(((1,), (1,)), ((), ())),
            preferred_element_type=jnp.float32)

    return pl.pallas_call(
        body,
        grid=(grid,),
        in_specs=[
            pl.BlockSpec((NC, blk, d), lambda j: (0, j, 0)),
            pl.BlockSpec((blk, d), lambda j: (j, 0)),
            pl.BlockSpec((blk, d), lambda j: (j, 0)),
            pl.BlockSpec((d, d), lambda j: (0, 0)),
            pl.BlockSpec((b, d), lambda j: (0, 0)),
        ],
        out_specs=pl.BlockSpec((b, blk), lambda j: (0, j)),
        out_shape=jax.ShapeDtypeStruct((b, n_pad), jnp.float32),
    )(partials, h1s, dinv2d, w2, img)


@jax.jit
def _run(x, edge_index, img, w1, w2):
    n, d = x.shape
    e = edge_index.shape[1]
    n_pad = ((n + NS * L - 1) // (NS * L)) * (NS * L)

    base = (-(-e // (NW * LI)) + 7) // 8 * 8
    er = base * NW
    e_pad = er * LI - e
    src_flat = jnp.pad(edge_index[0], (0, e_pad))
    dst_flat = jnp.pad(edge_index[1], (0, e_pad), constant_values=n)
    src2d = src_flat.reshape(er, LI)
    dst2d = dst_flat.reshape(er, LI)
    x_pad = jnp.pad(x, ((0, n_pad - n), (0, 0)))

    degp = _make_degree_kernel(er, n_pad)(dst2d)
    dinv2d, xs = _prep_tc(degp, x_pad)

    prop = _make_prop_kernel(er, n_pad, d, skew=72)
    p1 = prop(xs, src2d, dst2d)
    h1s = _layer1_tc(p1, xs, dinv2d, w1)
    p2 = prop(h1s, src2d, dst2d)
    scores_pad = _final_tc(p2, h1s, dinv2d, w2, img)
    return scores_pad[:, :n]


def kernel(x, edge_index, img, W1, W2):
    return _run(x, edge_index, img, W1, W2)
